# Initial kernel scaffold; baseline (speedup 1.0000x reference)
import jax
import jax.numpy as jnp
from jax.experimental import pallas as pl


def kernel(X, v_idx, e_idx, W, b):
    raise NotImplementedError("write your pallas kernel here")


# speedup vs baseline: 5.7447x; 5.7447x over previous
import dataclasses
import functools

import jax
import jax.numpy as jnp
from jax import lax
from jax.experimental import pallas as pl
from jax.experimental.pallas import tpu as pltpu
from jax.experimental.pallas import tpu_sc as plsc

N_NODES = 10000
N_NODES_PAD = 10240
N_HEDGES = 5000
N_HEDGES_PAD = 5120
N_EDGES = 320000
D = 128

NC = 2
NS = 16
NW = NC * NS
CH = 80
EPW = N_EDGES // NW
NCHUNK = EPW // CH

_MESH = plsc.VectorSubcoreMesh(core_axis_name="c", subcore_axis_name="s",
                               num_cores=NC, num_subcores=NS)


@functools.partial(
    pl.kernel,
    out_type=(
        jax.ShapeDtypeStruct((NW * N_HEDGES_PAD,), jnp.float32),
        jax.ShapeDtypeStruct((NW * N_NODES_PAD,), jnp.float32),
    ),
    mesh=_MESH,
    scratch_types=[
        pltpu.VMEM((EPW,), jnp.int32),
        pltpu.VMEM((EPW,), jnp.int32),
        pltpu.VMEM((N_HEDGES_PAD,), jnp.float32),
        pltpu.VMEM((N_NODES_PAD,), jnp.float32),
    ],
    compiler_params=dataclasses.replace(pltpu.CompilerParams(),
                                        needs_layout_passes=False),
)
def _sc_counts(vidx_hbm, eidx_hbm, cnt_e_hbm, cnt_v_hbm,
               vbuf, ebuf, hist_e, hist_v):
    c = lax.axis_index("c")
    s = lax.axis_index("s")
    w = c * NS + s
    z = jnp.zeros((16,), jnp.float32)
    one = jnp.ones((16,), jnp.float32)

    @pl.loop(0, N_HEDGES_PAD // 16)
    def _(i):
        hist_e[pl.ds(i * 16, 16)] = z

    @pl.loop(0, N_NODES_PAD // 16)
    def _(i):
        hist_v[pl.ds(i * 16, 16)] = z

    pltpu.sync_copy(vidx_hbm.at[pl.ds(w * EPW, EPW)], vbuf)
    pltpu.sync_copy(eidx_hbm.at[pl.ds(w * EPW, EPW)], ebuf)

    @pl.loop(0, EPW // 16)
    def _(k):
        plsc.addupdate_scatter(hist_e, [ebuf[pl.ds(k * 16, 16)]], one)
        plsc.addupdate_scatter(hist_v, [vbuf[pl.ds(k * 16, 16)]], one)

    pltpu.sync_copy(hist_e, cnt_e_hbm.at[pl.ds(w * N_HEDGES_PAD, N_HEDGES_PAD)])
    pltpu.sync_copy(hist_v, cnt_v_hbm.at[pl.ds(w * N_NODES_PAD, N_NODES_PAD)])


def _make_sc_hop(n_seg_pad, rows_per_sub):
    @functools.partial(
        pl.kernel,
        out_type=jax.ShapeDtypeStruct((NC, n_seg_pad, D), jnp.float32),
        mesh=_MESH,
        scratch_types=[
            pltpu.VMEM((CH,), jnp.int32),
            pltpu.VMEM((CH,), jnp.int32),
            pltpu.VMEM((CH, D), jnp.float32),
            pltpu.VMEM_SHARED((n_seg_pad, D), jnp.float32),
            pltpu.SemaphoreType.DMA,
        ],
    )
    def hop(table_hbm, gidx_hbm, sidx_hbm, z128_hbm, out_hbm,
            gbuf, sbuf, rows, acc, sem):
        c = lax.axis_index("c")
        s = lax.axis_index("s")
        pltpu.sync_copy(z128_hbm.at[pl.ds(0, rows_per_sub)],
                        acc.at[pl.ds(s * rows_per_sub, rows_per_sub)])
        plsc.subcore_barrier()

        base = (c * NS + s) * EPW

        @pl.loop(0, NCHUNK)
        def _(j):
            off = base + j * CH
            pltpu.sync_copy(gidx_hbm.at[pl.ds(off, CH)], gbuf)
            pltpu.sync_copy(sidx_hbm.at[pl.ds(off, CH)], sbuf)
            pltpu.async_copy(table_hbm.at[gbuf], rows, sem).wait()
            pltpu.sync_copy(rows, acc.at[sbuf], add=True)

        plsc.subcore_barrier()
        pltpu.sync_copy(acc.at[pl.ds(s * rows_per_sub, rows_per_sub)],
                        out_hbm.at[c, pl.ds(s * rows_per_sub, rows_per_sub)])

    return hop


_sc_hop_e = _make_sc_hop(N_HEDGES_PAD, N_HEDGES_PAD // NS)
_sc_hop_v = _make_sc_hop(N_NODES_PAD, N_NODES_PAD // NS)


def _mm_body(x_ref, w_ref, b_ref, o_ref):
    o_ref[...] = jnp.dot(x_ref[...], w_ref[...],
                         preferred_element_type=jnp.float32) + b_ref[...]


def _matmul(X, W, b):
    blk = 1000
    return pl.pallas_call(
        _mm_body,
        grid=(N_NODES // blk,),
        in_specs=[
            pl.BlockSpec((blk, D), lambda i: (i, 0)),
            pl.BlockSpec((D, D), lambda i: (0, 0)),
            pl.BlockSpec((1, D), lambda i: (0, 0)),
        ],
        out_specs=pl.BlockSpec((blk, D), lambda i: (i, 0)),
        out_shape=jax.ShapeDtypeStruct((N_NODES, D), jnp.float32),
    )(X, W, b.reshape(1, D))


def _norm_body(relu, p_ref, c_ref, o_ref):
    ssum = p_ref[0] + p_ref[1]
    cnt = jnp.sum(c_ref[...], axis=0)[:, None]
    res = ssum / jnp.maximum(cnt, 1.0)
    if relu:
        res = jnp.maximum(res, 0.0)
    o_ref[...] = res


def _combine_norm(parts, cnts, n_rows, blk, relu):
    return pl.pallas_call(
        functools.partial(_norm_body, relu),
        grid=(n_rows // blk,),
        in_specs=[
            pl.BlockSpec((NC, blk, D), lambda i: (0, i, 0)),
            pl.BlockSpec((NW, blk), lambda i: (0, i)),
        ],
        out_specs=pl.BlockSpec((blk, D), lambda i: (i, 0)),
        out_shape=jax.ShapeDtypeStruct((n_rows, D), jnp.float32),
    )(parts, cnts)


def kernel(X, v_idx, e_idx, W, b):
    v_idx = v_idx.astype(jnp.int32)
    e_idx = e_idx.astype(jnp.int32)
    z128 = jnp.zeros((640, D), jnp.float32)

    cnt_e, cnt_v = _sc_counts(v_idx, e_idx)
    cnt_e = cnt_e.reshape(NW, N_HEDGES_PAD)
    cnt_v = cnt_v.reshape(NW, N_NODES_PAD)
    Xt = _matmul(X, W, b)
    e_parts = _sc_hop_e(Xt, v_idx, e_idx, z128)
    e_feat = _combine_norm(e_parts, cnt_e, N_HEDGES_PAD, 1024, relu=False)
    v_parts = _sc_hop_v(e_feat, e_idx, v_idx, z128)
    out = _combine_norm(v_parts, cnt_v, N_NODES_PAD, 1024, relu=True)
    return out[:N_NODES]


# speedup vs baseline: 7.0472x; 1.2267x over previous
import dataclasses
import functools

import jax
import jax.numpy as jnp
from jax import lax
from jax.experimental import pallas as pl
from jax.experimental.pallas import tpu as pltpu
from jax.experimental.pallas import tpu_sc as plsc

N_NODES = 10000
N_NODES_PAD = 10240
N_HEDGES = 5000
N_HEDGES_PAD = 5120
N_EDGES = 320000
D = 128

NC = 2
NS = 16
NW = NC * NS
CH = 128
EPW = N_EDGES // NW
NCHUNK = 79
EPW_PAD = NCHUNK * CH
N_VACC = 10112
PAD_V = N_VACC - 1
PAD_E = N_HEDGES_PAD - 1

_MESH = plsc.VectorSubcoreMesh(core_axis_name="c", subcore_axis_name="s",
                               num_cores=NC, num_subcores=NS)


@functools.partial(
    pl.kernel,
    out_type=(
        jax.ShapeDtypeStruct((NW * N_HEDGES_PAD,), jnp.float32),
        jax.ShapeDtypeStruct((NW * N_NODES_PAD,), jnp.float32),
    ),
    mesh=_MESH,
    scratch_types=[
        pltpu.VMEM((EPW,), jnp.int32),
        pltpu.VMEM((EPW,), jnp.int32),
        pltpu.VMEM((N_HEDGES_PAD,), jnp.float32),
        pltpu.VMEM((N_NODES_PAD,), jnp.float32),
    ],
    compiler_params=dataclasses.replace(pltpu.CompilerParams(),
                                        needs_layout_passes=False),
)
def _sc_counts(vidx_hbm, eidx_hbm, cnt_e_hbm, cnt_v_hbm,
               vbuf, ebuf, hist_e, hist_v):
    c = lax.axis_index("c")
    s = lax.axis_index("s")
    w = c * NS + s
    z = jnp.zeros((16,), jnp.float32)
    one = jnp.ones((16,), jnp.float32)

    @pl.loop(0, N_HEDGES_PAD // 16)
    def _(i):
        hist_e[pl.ds(i * 16, 16)] = z

    @pl.loop(0, N_NODES_PAD // 16)
    def _(i):
        hist_v[pl.ds(i * 16, 16)] = z

    pltpu.sync_copy(vidx_hbm.at[pl.ds(w * EPW, EPW)], vbuf)
    pltpu.sync_copy(eidx_hbm.at[pl.ds(w * EPW, EPW)], ebuf)

    @pl.loop(0, EPW // 16)
    def _(k):
        plsc.addupdate_scatter(hist_e, [ebuf[pl.ds(k * 16, 16)]], one)
        plsc.addupdate_scatter(hist_v, [vbuf[pl.ds(k * 16, 16)]], one)

    pltpu.sync_copy(hist_e, cnt_e_hbm.at[pl.ds(w * N_HEDGES_PAD, N_HEDGES_PAD)])
    pltpu.sync_copy(hist_v, cnt_v_hbm.at[pl.ds(w * N_NODES_PAD, N_NODES_PAD)])


def _make_sc_hop(n_seg_pad, rows_per_sub, g_row, s_row, nbuf):
    @functools.partial(
        pl.kernel,
        out_type=jax.ShapeDtypeStruct((NC, n_seg_pad, D), jnp.float32),
        mesh=_MESH,
        scratch_types=[
            pltpu.VMEM((nbuf, 2, CH), jnp.int32),
            pltpu.VMEM((nbuf, CH, D), jnp.float32),
            pltpu.VMEM_SHARED((n_seg_pad, D), jnp.float32),
            pltpu.SemaphoreType.DMA,
            pltpu.SemaphoreType.DMA,
        ],
    )
    def hop(table_hbm, stk_hbm, z128_hbm, out_hbm,
            ibuf, rows, acc, isem, gsem):
        c = lax.axis_index("c")
        s = lax.axis_index("s")
        w = c * NS + s
        pltpu.sync_copy(z128_hbm.at[pl.ds(0, rows_per_sub)],
                        acc.at[pl.ds(s * rows_per_sub, rows_per_sub)])
        plsc.subcore_barrier()

        def i_issue(j, b):
            pltpu.async_copy(stk_hbm.at[w, j], ibuf.at[b], isem)

        def i_wait(j, b):
            pltpu.make_async_copy(stk_hbm.at[w, j], ibuf.at[b], isem).wait()

        def g_issue(j, b):
            pltpu.async_copy(table_hbm.at[ibuf.at[b, g_row]], rows.at[b], gsem)

        def g_wait(j, b):
            pltpu.make_async_copy(table_hbm.at[ibuf.at[b, g_row]],
                                  rows.at[b], gsem).wait()

        def s_do(j, b):
            pltpu.sync_copy(rows.at[b], acc.at[ibuf.at[b, s_row]], add=True)

        def step(j, b, has1, has2):
            if has1:
                i_wait(j + 1, (b + 1) % nbuf)
                g_issue(j + 1, (b + 1) % nbuf)
            if has2 and nbuf >= 3:
                i_issue(j + 2, (b + 2) % nbuf)
            g_wait(j, b)
            s_do(j, b)
            if has2 and nbuf == 2:
                i_issue(j + 2, (b + 2) % nbuf)

        i_issue(0, 0)
        i_issue(1, 1 % nbuf)
        i_wait(0, 0)
        g_issue(0, 0)

        m = ((NCHUNK - 2) // nbuf) * nbuf

        @pl.loop(0, m // nbuf)
        def _(g):
            for bp in range(nbuf):
                step(g * nbuf + bp, bp, True, True)

        for j in range(m, NCHUNK):
            step(j, j % nbuf, j + 1 < NCHUNK, j + 2 < NCHUNK)

        plsc.subcore_barrier()
        pltpu.sync_copy(acc.at[pl.ds(s * rows_per_sub, rows_per_sub)],
                        out_hbm.at[c, pl.ds(s * rows_per_sub, rows_per_sub)])

    return hop


_sc_hop_e = _make_sc_hop(N_HEDGES_PAD, N_HEDGES_PAD // NS, 0, 1, nbuf=4)
_sc_hop_v = _make_sc_hop(N_VACC, N_VACC // NS, 1, 0, nbuf=2)


def _mm_body(x_ref, w_ref, b_ref, o_ref):
    o_ref[...] = jnp.dot(x_ref[...], w_ref[...],
                         preferred_element_type=jnp.float32) + b_ref[...]


def _matmul(X, W, b):
    blk = 632
    return pl.pallas_call(
        _mm_body,
        grid=(N_VACC // blk,),
        in_specs=[
            pl.BlockSpec((blk, D), lambda i: (i, 0)),
            pl.BlockSpec((D, D), lambda i: (0, 0)),
            pl.BlockSpec((1, D), lambda i: (0, 0)),
        ],
        out_specs=pl.BlockSpec((blk, D), lambda i: (i, 0)),
        out_shape=jax.ShapeDtypeStruct((N_VACC, D), jnp.float32),
    )(X, W, b.reshape(1, D))


def _norm_body(relu, n_rows, p_ref, c_ref, o_ref):
    ssum = p_ref[0] + p_ref[1]
    cnt = jnp.sum(c_ref[...], axis=0)[:n_rows, None]
    res = ssum / jnp.maximum(cnt, 1.0)
    if relu:
        res = jnp.maximum(res, 0.0)
    o_ref[...] = res


def _combine_norm(parts, cnts, n_rows, relu):
    return pl.pallas_call(
        functools.partial(_norm_body, relu, n_rows),
        out_shape=jax.ShapeDtypeStruct((n_rows, D), jnp.float32),
    )(parts, cnts)


def kernel(X, v_idx, e_idx, W, b):
    v_idx = v_idx.astype(jnp.int32)
    e_idx = e_idx.astype(jnp.int32)
    z128 = jnp.zeros((640, D), jnp.float32)

    pad = ((0, 0), (0, EPW_PAD - EPW))
    vp = jnp.pad(v_idx.reshape(NW, EPW), pad, constant_values=PAD_V)
    ep = jnp.pad(e_idx.reshape(NW, EPW), pad, constant_values=PAD_E)
    stk = jnp.stack([vp.reshape(NW, NCHUNK, CH),
                     ep.reshape(NW, NCHUNK, CH)], axis=2)

    cnt_e, cnt_v = _sc_counts(v_idx, e_idx)
    cnt_e = cnt_e.reshape(NW, N_HEDGES_PAD)
    cnt_v = cnt_v.reshape(NW, N_NODES_PAD)

    Xp = jnp.pad(X, ((0, N_VACC - N_NODES), (0, 0)))
    Xt = _matmul(Xp, W, b)
    e_parts = _sc_hop_e(Xt, stk, z128)
    e_feat = _combine_norm(e_parts, cnt_e, N_HEDGES_PAD, relu=False)
    v_parts = _sc_hop_v(e_feat, stk, z128)
    out = _combine_norm(v_parts, cnt_v, N_VACC, relu=True)
    return out[:N_NODES]


# speedup vs baseline: 7.1622x; 1.0163x over previous
import dataclasses
import functools

import jax
import jax.numpy as jnp
from jax import lax
from jax.experimental import pallas as pl
from jax.experimental.pallas import tpu as pltpu
from jax.experimental.pallas import tpu_sc as plsc

N_NODES = 10000
N_NODES_PAD = 10240
N_HEDGES = 5000
N_HEDGES_PAD = 5120
N_EDGES = 320000
D = 128

NC = 2
NS = 16
NW = NC * NS
CH = 128
EPW = N_EDGES // NW
NCHUNK = 79
EPW_PAD = NCHUNK * CH
N_VACC = 10112
PAD_V = N_VACC - 1
PAD_E = N_HEDGES_PAD - 1

_MESH = plsc.VectorSubcoreMesh(core_axis_name="c", subcore_axis_name="s",
                               num_cores=NC, num_subcores=NS)


@functools.partial(
    pl.kernel,
    out_type=(
        jax.ShapeDtypeStruct((NW * N_HEDGES_PAD,), jnp.float32),
        jax.ShapeDtypeStruct((NW * N_NODES_PAD,), jnp.float32),
    ),
    mesh=_MESH,
    scratch_types=[
        pltpu.VMEM((EPW,), jnp.int32),
        pltpu.VMEM((EPW,), jnp.int32),
        pltpu.VMEM((N_HEDGES_PAD,), jnp.float32),
        pltpu.VMEM((N_NODES_PAD,), jnp.float32),
    ],
    compiler_params=dataclasses.replace(pltpu.CompilerParams(),
                                        needs_layout_passes=False),
)
def _sc_counts(vidx_hbm, eidx_hbm, cnt_e_hbm, cnt_v_hbm,
               vbuf, ebuf, hist_e, hist_v):
    c = lax.axis_index("c")
    s = lax.axis_index("s")
    w = c * NS + s
    z = jnp.zeros((16,), jnp.float32)
    one = jnp.ones((16,), jnp.float32)

    @pl.loop(0, N_HEDGES_PAD // 16)
    def _(i):
        hist_e[pl.ds(i * 16, 16)] = z

    @pl.loop(0, N_NODES_PAD // 16)
    def _(i):
        hist_v[pl.ds(i * 16, 16)] = z

    pltpu.sync_copy(vidx_hbm.at[pl.ds(w * EPW, EPW)], vbuf)
    pltpu.sync_copy(eidx_hbm.at[pl.ds(w * EPW, EPW)], ebuf)

    @pl.loop(0, EPW // 16)
    def _(k):
        plsc.addupdate_scatter(hist_e, [ebuf[pl.ds(k * 16, 16)]], one)
        plsc.addupdate_scatter(hist_v, [vbuf[pl.ds(k * 16, 16)]], one)

    pltpu.sync_copy(hist_e, cnt_e_hbm.at[pl.ds(w * N_HEDGES_PAD, N_HEDGES_PAD)])
    pltpu.sync_copy(hist_v, cnt_v_hbm.at[pl.ds(w * N_NODES_PAD, N_NODES_PAD)])


def _make_sc_hop(n_seg_pad, rows_per_sub, g_row, s_row, nbuf, async_scatter):
    @functools.partial(
        pl.kernel,
        out_type=jax.ShapeDtypeStruct((NC, n_seg_pad, D), jnp.float32),
        mesh=_MESH,
        scratch_types=[
            pltpu.VMEM((nbuf, 2, CH), jnp.int32),
            pltpu.VMEM((nbuf, CH, D), jnp.float32),
            pltpu.VMEM_SHARED((n_seg_pad, D), jnp.float32),
            pltpu.SemaphoreType.DMA,
            pltpu.SemaphoreType.DMA,
            pltpu.SemaphoreType.DMA,
        ],
    )
    def hop(table_hbm, stk_hbm, z128_hbm, out_hbm,
            ibuf, rows, acc, isem, gsem, ssem):
        c = lax.axis_index("c")
        s = lax.axis_index("s")
        w = c * NS + s
        pltpu.sync_copy(z128_hbm.at[pl.ds(0, rows_per_sub)],
                        acc.at[pl.ds(s * rows_per_sub, rows_per_sub)])
        plsc.subcore_barrier()

        def i_issue(j, b):
            pltpu.async_copy(stk_hbm.at[w, j], ibuf.at[b], isem)

        def i_wait(j, b):
            pltpu.make_async_copy(stk_hbm.at[w, j], ibuf.at[b], isem).wait()

        def g_issue(j, b):
            pltpu.async_copy(table_hbm.at[ibuf.at[b, g_row]], rows.at[b], gsem)

        def g_wait(j, b):
            pltpu.make_async_copy(table_hbm.at[ibuf.at[b, g_row]],
                                  rows.at[b], gsem).wait()

        def s_issue(j, b):
            pltpu.async_copy(rows.at[b], acc.at[ibuf.at[b, s_row]], ssem,
                             add=True)

        def s_wait(j, b):
            pltpu.make_async_copy(rows.at[b], acc.at[ibuf.at[b, s_row]],
                                  ssem).wait()

        def s_do(j, b):
            pltpu.sync_copy(rows.at[b], acc.at[ibuf.at[b, s_row]], add=True)

        def step(j, b, has1, has2, drain):
            if async_scatter and drain:
                s_wait(j - 2, (b + 2) % nbuf)
            if has1:
                i_wait(j + 1, (b + 1) % nbuf)
                g_issue(j + 1, (b + 1) % nbuf)
            if has2 and nbuf >= 3:
                i_issue(j + 2, (b + 2) % nbuf)
            g_wait(j, b)
            if async_scatter:
                s_issue(j, b)
            else:
                s_do(j, b)
            if has2 and nbuf == 2:
                i_issue(j + 2, (b + 2) % nbuf)

        i_issue(0, 0)
        i_issue(1, 1 % nbuf)
        i_wait(0, 0)
        g_issue(0, 0)

        if async_scatter:
            step(0, 0, True, True, False)
            step(1, 1 % nbuf, True, True, False)
            m = ((NCHUNK - 4) // nbuf) * nbuf

            @pl.loop(0, m // nbuf)
            def _(g):
                for bp in range(nbuf):
                    j = 2 + g * nbuf + bp
                    step(j, (2 + bp) % nbuf, True, True, True)

            for j in range(2 + m, NCHUNK):
                step(j, j % nbuf, j + 1 < NCHUNK, j + 2 < NCHUNK, True)
            s_wait(NCHUNK - 2, (NCHUNK - 2) % nbuf)
            s_wait(NCHUNK - 1, (NCHUNK - 1) % nbuf)
        else:
            m = ((NCHUNK - 2) // nbuf) * nbuf

            @pl.loop(0, m // nbuf)
            def _(g):
                for bp in range(nbuf):
                    step(g * nbuf + bp, bp, True, True, False)

            for j in range(m, NCHUNK):
                step(j, j % nbuf, j + 1 < NCHUNK, j + 2 < NCHUNK, False)

        plsc.subcore_barrier()
        pltpu.sync_copy(acc.at[pl.ds(s * rows_per_sub, rows_per_sub)],
                        out_hbm.at[c, pl.ds(s * rows_per_sub, rows_per_sub)])

    return hop


_sc_hop_e = _make_sc_hop(N_HEDGES_PAD, N_HEDGES_PAD // NS, 0, 1, nbuf=4,
                         async_scatter=True)
_sc_hop_v = _make_sc_hop(N_VACC, N_VACC // NS, 1, 0, nbuf=2,
                         async_scatter=False)


def _mm_body(x_ref, w_ref, b_ref, o_ref):
    o_ref[...] = jnp.dot(x_ref[...], w_ref[...],
                         preferred_element_type=jnp.float32) + b_ref[...]


def _matmul(X, W, b):
    blk = 632
    return pl.pallas_call(
        _mm_body,
        grid=(N_VACC // blk,),
        in_specs=[
            pl.BlockSpec((blk, D), lambda i: (i, 0)),
            pl.BlockSpec((D, D), lambda i: (0, 0)),
            pl.BlockSpec((1, D), lambda i: (0, 0)),
        ],
        out_specs=pl.BlockSpec((blk, D), lambda i: (i, 0)),
        out_shape=jax.ShapeDtypeStruct((N_VACC, D), jnp.float32),
    )(X, W, b.reshape(1, D))


def _norm_body(relu, n_rows, p_ref, c_ref, o_ref):
    ssum = p_ref[0] + p_ref[1]
    cnt = jnp.sum(c_ref[...], axis=0)[:n_rows, None]
    res = ssum / jnp.maximum(cnt, 1.0)
    if relu:
        res = jnp.maximum(res, 0.0)
    o_ref[...] = res


def _combine_norm(parts, cnts, n_rows, relu):
    return pl.pallas_call(
        functools.partial(_norm_body, relu, n_rows),
        out_shape=jax.ShapeDtypeStruct((n_rows, D), jnp.float32),
    )(parts, cnts)


def kernel(X, v_idx, e_idx, W, b):
    v_idx = v_idx.astype(jnp.int32)
    e_idx = e_idx.astype(jnp.int32)
    z128 = jnp.zeros((640, D), jnp.float32)

    pad = ((0, 0), (0, EPW_PAD - EPW))
    vp = jnp.pad(v_idx.reshape(NW, EPW), pad, constant_values=PAD_V)
    ep = jnp.pad(e_idx.reshape(NW, EPW), pad, constant_values=PAD_E)
    stk = jnp.stack([vp.reshape(NW, NCHUNK, CH),
                     ep.reshape(NW, NCHUNK, CH)], axis=2)

    cnt_e, cnt_v = _sc_counts(v_idx, e_idx)
    cnt_e = cnt_e.reshape(NW, N_HEDGES_PAD)
    cnt_v = cnt_v.reshape(NW, N_NODES_PAD)

    Xp = jnp.pad(X, ((0, N_VACC - N_NODES), (0, 0)))
    Xt = _matmul(Xp, W, b)
    e_parts = _sc_hop_e(Xt, stk, z128)
    e_feat = _combine_norm(e_parts, cnt_e, N_HEDGES_PAD, relu=False)
    v_parts = _sc_hop_v(e_feat, stk, z128)
    out = _combine_norm(v_parts, cnt_v, N_VACC, relu=True)
    return out[:N_NODES]
